# baseline (speedup 1.0000x reference)
import jax
import jax.numpy as jnp
from jax.experimental import pallas as pl
from jax.experimental.pallas import tpu as pltpu

EPS = 1e-5

RB = 512
CB = 2048
N_ROWS = 4608
N_CODES = 8192
NCB = N_CODES // CB


def _prep_body(u_ref, es_ref, ebf_ref, e2_ref, colf_ref):
    emb = es_ref[...] / jnp.maximum(u_ref[...], EPS)
    ebf_ref[...] = emb.astype(jnp.bfloat16)
    e2_ref[...] = jnp.sum(emb * emb, axis=1)[None, :]
    colf_ref[...] = jax.lax.broadcasted_iota(
        jnp.int32, (1, N_CODES), 1).astype(jnp.float32)


def _main_body(x_ref, ebf_ref, e2_ref, colf_ref, out_ref,
               xbf_ref, x2_ref, bv_ref, bi_ref):
    j = pl.program_id(1)

    @pl.when(j == 0)
    def _():
        xb = x_ref[...]
        x2_ref[...] = jnp.sum(xb * xb, axis=1, keepdims=True)
        xbf_ref[...] = (xb * -2.0).astype(jnp.bfloat16)

    s = jax.lax.dot_general(
        xbf_ref[...], ebf_ref[...],
        dimension_numbers=(((1,), (1,)), ((), ())),
        preferred_element_type=jnp.float32,
    )
    d2 = (x2_ref[...] + e2_ref[...]) + s

    lmin = jnp.min(d2, axis=1, keepdims=True)
    lidx = jnp.min(jnp.where(d2 == lmin, colf_ref[...], jnp.float32(1e30)),
                   axis=1, keepdims=True)

    @pl.when(j == 0)
    def _():
        bv_ref[...] = lmin
        bi_ref[...] = lidx

    @pl.when(j > 0)
    def _():
        better = lmin < bv_ref[...]
        bv_ref[...] = jnp.where(better, lmin, bv_ref[...])
        bi_ref[...] = jnp.where(better, lidx, bi_ref[...])

    @pl.when(j == NCB - 1)
    def _():
        out_ref[...] = bi_ref[...].astype(jnp.int32)


def kernel(x, cluster_usage, embedding_sum):
    B, D, T = x.shape
    xf = jnp.transpose(x, (0, 2, 1)).reshape(B * T, D)
    usage = cluster_usage.reshape(N_CODES, 1)

    ebf, e2, colf = pl.pallas_call(
        _prep_body,
        out_shape=(
            jax.ShapeDtypeStruct((N_CODES, D), jnp.bfloat16),
            jax.ShapeDtypeStruct((1, N_CODES), jnp.float32),
            jax.ShapeDtypeStruct((1, N_CODES), jnp.float32),
        ),
    )(usage, embedding_sum)

    codes = pl.pallas_call(
        _main_body,
        grid=(N_ROWS // RB, NCB),
        in_specs=[
            pl.BlockSpec((RB, D), lambda i, j: (i, 0)),
            pl.BlockSpec((CB, D), lambda i, j: (j, 0)),
            pl.BlockSpec((1, CB), lambda i, j: (0, j)),
            pl.BlockSpec((1, CB), lambda i, j: (0, j)),
        ],
        out_specs=pl.BlockSpec((RB, 1), lambda i, j: (i, 0)),
        out_shape=jax.ShapeDtypeStruct((N_ROWS, 1), jnp.int32),
        scratch_shapes=[
            pltpu.VMEM((RB, D), jnp.bfloat16),
            pltpu.VMEM((RB, 1), jnp.float32),
            pltpu.VMEM((RB, 1), jnp.float32),
            pltpu.VMEM((RB, 1), jnp.float32),
        ],
    )(xf, ebf, e2, colf)

    return codes.reshape(B, 1, T)
